# baseline (speedup 1.0000x reference)
import functools

import jax
import jax.numpy as jnp
from jax import lax
from jax.experimental import pallas as pl
from jax.experimental.pallas import tpu as pltpu
from jax.experimental.pallas import tpu_sc as plsc

L = 16
NC = 2
NS = 16
NW = NC * NS
R = 4
CHUNK = 4096


def _sc_partials(xs, indices, values, tidx, tval, bias1d, n_bat, n_in, n_out):
    nbg = n_bat // R
    splits = NW // nbg
    nnz = values.shape[0]
    n_full = nnz // CHUNK
    q = n_full // splits
    n_pairs = q // 2

    mesh = plsc.VectorSubcoreMesh(core_axis_name="c", subcore_axis_name="s")

    @functools.partial(
        pl.kernel,
        out_type=jax.ShapeDtypeStruct((splits * n_bat, n_out), jnp.float32),
        mesh=mesh,
        compiler_params=pltpu.CompilerParams(needs_layout_passes=False),
        scratch_types=[
            pltpu.VMEM((n_in,), jnp.int32),
            pltpu.VMEM((n_in,), jnp.int32),
            pltpu.VMEM((n_out,), jnp.float32),
            pltpu.VMEM((n_out,), jnp.float32),
            pltpu.VMEM((n_out,), jnp.float32),
            pltpu.VMEM((n_out,), jnp.float32),
            pltpu.VMEM((2, CHUNK), jnp.int32),
            pltpu.VMEM((CHUNK,), jnp.float32),
            pltpu.VMEM((2, CHUNK), jnp.int32),
            pltpu.VMEM((CHUNK,), jnp.float32),
            pltpu.SemaphoreType.DMA,
            pltpu.SemaphoreType.DMA,
        ],
    )
    def spmm(xs_hbm, idx_hbm, val_hbm, tidx_hbm, tval_hbm, bias_hbm, out_hbm,
             x01, x23, a0, a1, a2, a3, ib0, vb0, ib1, vb1, sem0, sem1):
        ibufs, vbufs, sems = (ib0, ib1), (vb0, vb1), (sem0, sem1)
        accs = (a0, a1, a2, a3)
        wid = lax.axis_index("c") * NS + lax.axis_index("s")
        bg = wid % nbg
        sp = wid // nbg
        row0 = bg * R

        half = jnp.full((L,), 0x7FFF, jnp.int32)
        one = jnp.full((L,), 1, jnp.int32)
        hi_mask = jnp.full((L,), -65536, jnp.int32)

        def rnd(f):
            b = plsc.bitcast(f, jnp.int32)
            return b + half + lax.bitwise_and(lax.shift_right_logical(b, 16), one)

        for dst_ref, j in ((x01, 0), (x23, 2)):
            pltpu.sync_copy(xs_hbm.at[row0 + j], a0)
            pltpu.sync_copy(xs_hbm.at[row0 + j + 1], a1)

            @plsc.parallel_loop(0, n_in, step=L, unroll=4)
            def pack(o):
                hi = lax.bitwise_and(rnd(a0[pl.ds(o, L)]), hi_mask)
                lo = lax.shift_right_logical(rnd(a1[pl.ds(o, L)]), 16)
                dst_ref[pl.ds(o, L)] = lax.bitwise_or(hi, lo)

        @pl.when(sp == 0)
        def _():
            for a in accs:
                pltpu.sync_copy(bias_hbm, a)

        @pl.when(sp != 0)
        def _():
            zv = jnp.zeros((L,), jnp.float32)

            @plsc.parallel_loop(0, n_out, step=L, unroll=4)
            def zbody(o):
                for a in accs:
                    a[pl.ds(o, L)] = zv

        def start(ci, b):
            off = (ci * splits + sp) * CHUNK
            pltpu.async_copy(idx_hbm.at[:, pl.ds(off, CHUNK)], ibufs[b], sems[b])
            pltpu.async_copy(val_hbm.at[pl.ds(off, CHUNK)], vbufs[b], sems[b])

        def wait(ci, b):
            off = (ci * splits + sp) * CHUNK
            pltpu.make_async_copy(idx_hbm.at[:, pl.ds(off, CHUNK)], ibufs[b], sems[b]).wait()
            pltpu.make_async_copy(val_hbm.at[pl.ds(off, CHUNK)], vbufs[b], sems[b]).wait()

        def compute(b):
            ibuf, vbuf = ibufs[b], vbufs[b]

            @plsc.parallel_loop(0, CHUNK, step=L, unroll=16)
            def grp(o):
                s_idx = ibuf[0, pl.ds(o, L)]
                d_idx = ibuf[1, pl.ds(o, L)]
                v = vbuf[pl.ds(o, L)]
                g01 = plsc.load_gather(x01, [s_idx])
                g23 = plsc.load_gather(x23, [s_idx])
                r0 = plsc.bitcast(lax.bitwise_and(g01, hi_mask), jnp.float32)
                r1 = plsc.bitcast(lax.shift_left(g01, 16), jnp.float32)
                r2 = plsc.bitcast(lax.bitwise_and(g23, hi_mask), jnp.float32)
                r3 = plsc.bitcast(lax.shift_left(g23, 16), jnp.float32)
                plsc.addupdate_scatter(a0, [d_idx], r0 * v)
                plsc.addupdate_scatter(a1, [d_idx], r1 * v)
                plsc.addupdate_scatter(a2, [d_idx], r2 * v)
                plsc.addupdate_scatter(a3, [d_idx], r3 * v)

        @pl.when(jnp.bool_(n_pairs > 0))
        def _():
            start(0, 0)

            def chunk_pair(k, c):
                ci = k * 2
                start(ci + 1, 1)
                wait(ci, 0)
                compute(0)

                @pl.when(ci + 2 < n_pairs * 2)
                def _():
                    start(ci + 2, 0)

                wait(ci + 1, 1)
                compute(1)
                return c

            lax.fori_loop(0, n_pairs, chunk_pair, 0)

        for j in range(2 * n_pairs, (n_full + splits - 1) // splits):
            @pl.when(j * splits + sp < n_full)
            def _():
                b = j % 2
                start(j, b)
                wait(j, b)
                compute(b)

        @pl.when(sp == splits - 1)
        def _():
            pltpu.async_copy(tidx_hbm, ib0, sem0)
            pltpu.async_copy(tval_hbm, vb0, sem0)
            pltpu.make_async_copy(tidx_hbm, ib0, sem0).wait()
            pltpu.make_async_copy(tval_hbm, vb0, sem0).wait()
            compute(0)

        for j, a in enumerate(accs):
            pltpu.sync_copy(a, out_hbm.at[sp * n_bat + row0 + j])

    return spmm(xs, indices, values, tidx, tval, bias1d)


def _tc_reduce(partials, splits, n_bat, n_out):
    blk = 2048

    def body(p_ref, o_ref):
        o_ref[...] = jnp.sum(p_ref[...], axis=0)

    return pl.pallas_call(
        body,
        grid=(n_out // blk,),
        in_specs=[pl.BlockSpec((splits, n_bat, blk), lambda i: (0, 0, i))],
        out_specs=pl.BlockSpec((n_bat, blk), lambda i: (0, i)),
        out_shape=jax.ShapeDtypeStruct((n_bat, n_out), jnp.float32),
    )(partials)


def kernel(x, indices, values, bias):
    n_bat, n_in = x.shape[0], x.shape[1]
    n_out = bias.shape[0]
    nnz = values.shape[0]

    nbg = n_bat // R
    splits = NW // nbg

    xs = x.reshape(n_bat, n_in)
    bias1d = bias.reshape(n_out)

    n_full = nnz // CHUNK
    t = nnz - n_full * CHUNK
    tidx = jnp.pad(lax.slice(indices, (0, nnz - t), (2, nnz)),
                   ((0, 0), (0, CHUNK - t)))
    tval = jnp.pad(lax.slice(values, (nnz - t,), (nnz,)), (0, CHUNK - t))

    partials = _sc_partials(xs, indices, values, tidx, tval, bias1d,
                            n_bat, n_in, n_out)
    partials = partials.reshape(splits, n_bat, n_out)
    out = _tc_reduce(partials, splits, n_bat, n_out)
    return out[..., None]


# speedup vs baseline: 1.0989x; 1.0989x over previous
import functools

import jax
import jax.numpy as jnp
from jax import lax
from jax.experimental import pallas as pl
from jax.experimental.pallas import tpu as pltpu
from jax.experimental.pallas import tpu_sc as plsc

L = 16
NC = 2
NS = 16
NW = NC * NS
R = 4
CHUNK = 4096


def _sc_partials(xs, indices, values, tidx, tval, bias1d, n_bat, n_in, n_out):
    nbg = n_bat // R
    splits = NW // nbg
    nnz = values.shape[0]
    n_full = nnz // CHUNK
    q = n_full // splits
    n_pairs = q // 2

    mesh = plsc.VectorSubcoreMesh(core_axis_name="c", subcore_axis_name="s")

    @functools.partial(
        pl.kernel,
        out_type=jax.ShapeDtypeStruct((splits * n_bat, n_out), jnp.float32),
        mesh=mesh,
        compiler_params=pltpu.CompilerParams(needs_layout_passes=False),
        scratch_types=[
            pltpu.VMEM((n_in,), jnp.int32),
            pltpu.VMEM((n_in,), jnp.int32),
            pltpu.VMEM((n_out,), jnp.float32),
            pltpu.VMEM((n_out,), jnp.float32),
            pltpu.VMEM((n_out,), jnp.float32),
            pltpu.VMEM((n_out,), jnp.float32),
            pltpu.VMEM((2, CHUNK), jnp.int32),
            pltpu.VMEM((CHUNK,), jnp.float32),
            pltpu.VMEM((2, CHUNK), jnp.int32),
            pltpu.VMEM((CHUNK,), jnp.float32),
            pltpu.SemaphoreType.DMA,
            pltpu.SemaphoreType.DMA,
            pltpu.SemaphoreType.DMA,
        ],
    )
    def spmm(xs_hbm, idx_hbm, val_hbm, tidx_hbm, tval_hbm, bias_hbm, out_hbm,
             x01, x23, a0, a1, a2, a3, ib0, vb0, ib1, vb1, sem0, sem1, sem2):
        ibufs, vbufs, sems = (ib0, ib1), (vb0, vb1), (sem0, sem1)
        accs = (a0, a1, a2, a3)
        wid = lax.axis_index("c") * NS + lax.axis_index("s")
        bg = wid % nbg
        sp = wid // nbg
        row0 = bg * R

        def start(ci, b):
            off = (ci * splits + sp) * CHUNK
            pltpu.async_copy(idx_hbm.at[:, pl.ds(off, CHUNK)], ibufs[b], sems[b])
            pltpu.async_copy(val_hbm.at[pl.ds(off, CHUNK)], vbufs[b], sems[b])

        def wait(ci, b):
            off = (ci * splits + sp) * CHUNK
            pltpu.make_async_copy(idx_hbm.at[:, pl.ds(off, CHUNK)], ibufs[b], sems[b]).wait()
            pltpu.make_async_copy(val_hbm.at[pl.ds(off, CHUNK)], vbufs[b], sems[b]).wait()

        @pl.when(jnp.bool_(n_pairs > 0))
        def _():
            start(0, 0)
            start(1, 1)

        half = jnp.full((L,), 0x7FFF, jnp.int32)
        one = jnp.full((L,), 1, jnp.int32)
        hi_mask = jnp.full((L,), -65536, jnp.int32)

        def rnd(f):
            b = plsc.bitcast(f, jnp.int32)
            return b + half + lax.bitwise_and(lax.shift_right_logical(b, 16), one)

        for j, a in enumerate(accs):
            pltpu.async_copy(xs_hbm.at[row0 + j], a, sem2)
        for j, a in enumerate(accs):
            pltpu.make_async_copy(xs_hbm.at[row0 + j], a, sem2).wait()

        for dst_ref, srcs in ((x01, (a0, a1)), (x23, (a2, a3))):
            @plsc.parallel_loop(0, n_in, step=L, unroll=4)
            def pack(o):
                hi = lax.bitwise_and(rnd(srcs[0][pl.ds(o, L)]), hi_mask)
                lo = lax.shift_right_logical(rnd(srcs[1][pl.ds(o, L)]), 16)
                dst_ref[pl.ds(o, L)] = lax.bitwise_or(hi, lo)

        @pl.when(sp == 0)
        def _():
            for a in accs:
                pltpu.sync_copy(bias_hbm, a)

        @pl.when(sp != 0)
        def _():
            zv = jnp.zeros((L,), jnp.float32)

            @plsc.parallel_loop(0, n_out, step=L, unroll=4)
            def zbody(o):
                for a in accs:
                    a[pl.ds(o, L)] = zv

        def compute(b):
            ibuf, vbuf = ibufs[b], vbufs[b]

            @plsc.parallel_loop(0, CHUNK, step=L, unroll=8)
            def grp(o):
                s_idx = ibuf[0, pl.ds(o, L)]
                d_idx = ibuf[1, pl.ds(o, L)]
                v = vbuf[pl.ds(o, L)]
                g01 = plsc.load_gather(x01, [s_idx])
                g23 = plsc.load_gather(x23, [s_idx])
                r0 = plsc.bitcast(lax.bitwise_and(g01, hi_mask), jnp.float32)
                r1 = plsc.bitcast(lax.shift_left(g01, 16), jnp.float32)
                r2 = plsc.bitcast(lax.bitwise_and(g23, hi_mask), jnp.float32)
                r3 = plsc.bitcast(lax.shift_left(g23, 16), jnp.float32)
                plsc.addupdate_scatter(a0, [d_idx], r0 * v)
                plsc.addupdate_scatter(a1, [d_idx], r1 * v)
                plsc.addupdate_scatter(a2, [d_idx], r2 * v)
                plsc.addupdate_scatter(a3, [d_idx], r3 * v)

        @pl.when(jnp.bool_(n_pairs > 0))
        def _():
            def chunk_pair(k, c):
                ci = k * 2
                wait(ci, 0)
                compute(0)

                @pl.when(ci + 2 < n_pairs * 2)
                def _():
                    start(ci + 2, 0)

                wait(ci + 1, 1)
                compute(1)

                @pl.when(ci + 3 < n_pairs * 2)
                def _():
                    start(ci + 3, 1)

                return c

            lax.fori_loop(0, n_pairs, chunk_pair, 0)

        for j in range(2 * n_pairs, (n_full + splits - 1) // splits):
            @pl.when(j * splits + sp < n_full)
            def _():
                b = j % 2
                start(j, b)
                wait(j, b)
                compute(b)

        @pl.when(sp == splits - 1)
        def _():
            pltpu.async_copy(tidx_hbm, ib0, sem0)
            pltpu.async_copy(tval_hbm, vb0, sem0)
            pltpu.make_async_copy(tidx_hbm, ib0, sem0).wait()
            pltpu.make_async_copy(tval_hbm, vb0, sem0).wait()
            compute(0)

        for j, a in enumerate(accs):
            pltpu.async_copy(a, out_hbm.at[sp * n_bat + row0 + j], sem2)
        for j, a in enumerate(accs):
            pltpu.make_async_copy(a, out_hbm.at[sp * n_bat + row0 + j], sem2).wait()

    return spmm(xs, indices, values, tidx, tval, bias1d)


def _tc_reduce(partials, splits, n_bat, n_out):
    blk = 2048

    def body(p_ref, o_ref):
        o_ref[...] = jnp.sum(p_ref[...], axis=0)

    return pl.pallas_call(
        body,
        grid=(n_out // blk,),
        in_specs=[pl.BlockSpec((splits, n_bat, blk), lambda i: (0, 0, i))],
        out_specs=pl.BlockSpec((n_bat, blk), lambda i: (0, i)),
        out_shape=jax.ShapeDtypeStruct((n_bat, n_out), jnp.float32),
    )(partials)


def kernel(x, indices, values, bias):
    n_bat, n_in = x.shape[0], x.shape[1]
    n_out = bias.shape[0]
    nnz = values.shape[0]

    nbg = n_bat // R
    splits = NW // nbg

    xs = x.reshape(n_bat, n_in)
    bias1d = bias.reshape(n_out)

    n_full = nnz // CHUNK
    t = nnz - n_full * CHUNK
    tidx = jnp.pad(lax.slice(indices, (0, nnz - t), (2, nnz)),
                   ((0, 0), (0, CHUNK - t)))
    tval = jnp.pad(lax.slice(values, (nnz - t,), (nnz,)), (0, CHUNK - t))

    partials = _sc_partials(xs, indices, values, tidx, tval, bias1d,
                            n_bat, n_in, n_out)
    partials = partials.reshape(splits, n_bat, n_out)
    out = _tc_reduce(partials, splits, n_bat, n_out)
    return out[..., None]


# speedup vs baseline: 1.1037x; 1.0043x over previous
import functools

import jax
import jax.numpy as jnp
from jax import lax
from jax.experimental import pallas as pl
from jax.experimental.pallas import tpu as pltpu
from jax.experimental.pallas import tpu_sc as plsc

L = 16
NC = 2
NS = 16
NW = NC * NS
R = 4
CHUNK = 4096


def _sc_partials(xs, indices, values, tidx, tval, bias1d, n_bat, n_in, n_out):
    nbg = n_bat // R
    splits = NW // nbg
    nnz = values.shape[0]
    n_full = nnz // CHUNK
    q = n_full // splits
    n_pairs = q // 2

    mesh = plsc.VectorSubcoreMesh(core_axis_name="c", subcore_axis_name="s")

    @functools.partial(
        pl.kernel,
        out_type=jax.ShapeDtypeStruct((splits * n_bat, n_out), jnp.float32),
        mesh=mesh,
        compiler_params=pltpu.CompilerParams(needs_layout_passes=False),
        scratch_types=[
            pltpu.VMEM((n_in,), jnp.int32),
            pltpu.VMEM((n_in,), jnp.int32),
            pltpu.VMEM((n_out,), jnp.float32),
            pltpu.VMEM((n_out,), jnp.float32),
            pltpu.VMEM((n_out,), jnp.float32),
            pltpu.VMEM((n_out,), jnp.float32),
            pltpu.VMEM((2, CHUNK), jnp.int32),
            pltpu.VMEM((CHUNK,), jnp.float32),
            pltpu.VMEM((2, CHUNK), jnp.int32),
            pltpu.VMEM((CHUNK,), jnp.float32),
            pltpu.SemaphoreType.DMA,
            pltpu.SemaphoreType.DMA,
            pltpu.SemaphoreType.DMA,
        ],
    )
    def spmm(xs_hbm, idx_hbm, val_hbm, tidx_hbm, tval_hbm, bias_hbm, out_hbm,
             x01, x23, a0, a1, a2, a3, ib0, vb0, ib1, vb1, sem0, sem1, sem2):
        ibufs, vbufs, sems = (ib0, ib1), (vb0, vb1), (sem0, sem1)
        accs = (a0, a1, a2, a3)
        wid = lax.axis_index("c") * NS + lax.axis_index("s")
        bg = wid % nbg
        sp = wid // nbg
        row0 = bg * R

        def start(ci, b):
            off = (ci * splits + sp) * CHUNK
            pltpu.async_copy(idx_hbm.at[:, pl.ds(off, CHUNK)], ibufs[b], sems[b])
            pltpu.async_copy(val_hbm.at[pl.ds(off, CHUNK)], vbufs[b], sems[b])

        def wait(ci, b):
            off = (ci * splits + sp) * CHUNK
            pltpu.make_async_copy(idx_hbm.at[:, pl.ds(off, CHUNK)], ibufs[b], sems[b]).wait()
            pltpu.make_async_copy(val_hbm.at[pl.ds(off, CHUNK)], vbufs[b], sems[b]).wait()

        @pl.when(jnp.bool_(n_pairs > 0))
        def _():
            start(0, 0)
            start(1, 1)

        half = jnp.full((L,), 0x7FFF, jnp.int32)
        one = jnp.full((L,), 1, jnp.int32)
        hi_mask = jnp.full((L,), -65536, jnp.int32)

        def rnd(f):
            b = plsc.bitcast(f, jnp.int32)
            return b + half + lax.bitwise_and(lax.shift_right_logical(b, 16), one)

        for j, a in enumerate(accs):
            pltpu.async_copy(xs_hbm.at[row0 + j], a, sem2)
        for j, a in enumerate(accs):
            pltpu.make_async_copy(xs_hbm.at[row0 + j], a, sem2).wait()

        for dst_ref, srcs in ((x01, (a0, a1)), (x23, (a2, a3))):
            @plsc.parallel_loop(0, n_in, step=L, unroll=4)
            def pack(o):
                hi = lax.bitwise_and(rnd(srcs[0][pl.ds(o, L)]), hi_mask)
                lo = lax.shift_right_logical(rnd(srcs[1][pl.ds(o, L)]), 16)
                dst_ref[pl.ds(o, L)] = lax.bitwise_or(hi, lo)

        @pl.when(sp == 0)
        def _():
            for a in accs:
                pltpu.sync_copy(bias_hbm, a)

        @pl.when(sp != 0)
        def _():
            zv = jnp.zeros((L,), jnp.float32)

            @plsc.parallel_loop(0, n_out, step=L, unroll=4)
            def zbody(o):
                for a in accs:
                    a[pl.ds(o, L)] = zv

        def compute(b):
            ibuf, vbuf = ibufs[b], vbufs[b]

            @plsc.parallel_loop(0, CHUNK, step=L, unroll=12)
            def grp(o):
                s_idx = ibuf[0, pl.ds(o, L)]
                d_idx = ibuf[1, pl.ds(o, L)]
                v = vbuf[pl.ds(o, L)]
                g01 = plsc.load_gather(x01, [s_idx])
                g23 = plsc.load_gather(x23, [s_idx])
                r0 = plsc.bitcast(lax.bitwise_and(g01, hi_mask), jnp.float32)
                r1 = plsc.bitcast(lax.shift_left(g01, 16), jnp.float32)
                r2 = plsc.bitcast(lax.bitwise_and(g23, hi_mask), jnp.float32)
                r3 = plsc.bitcast(lax.shift_left(g23, 16), jnp.float32)
                plsc.addupdate_scatter(a0, [d_idx], r0 * v)
                plsc.addupdate_scatter(a1, [d_idx], r1 * v)
                plsc.addupdate_scatter(a2, [d_idx], r2 * v)
                plsc.addupdate_scatter(a3, [d_idx], r3 * v)

        @pl.when(jnp.bool_(n_pairs > 0))
        def _():
            def chunk_pair(k, c):
                ci = k * 2
                wait(ci, 0)
                compute(0)

                @pl.when(ci + 2 < n_pairs * 2)
                def _():
                    start(ci + 2, 0)

                wait(ci + 1, 1)
                compute(1)

                @pl.when(ci + 3 < n_pairs * 2)
                def _():
                    start(ci + 3, 1)

                return c

            lax.fori_loop(0, n_pairs, chunk_pair, 0)

        for j in range(2 * n_pairs, (n_full + splits - 1) // splits):
            @pl.when(j * splits + sp < n_full)
            def _():
                b = j % 2
                start(j, b)
                wait(j, b)
                compute(b)

        @pl.when(sp == splits - 1)
        def _():
            pltpu.async_copy(tidx_hbm, ib0, sem0)
            pltpu.async_copy(tval_hbm, vb0, sem0)
            pltpu.make_async_copy(tidx_hbm, ib0, sem0).wait()
            pltpu.make_async_copy(tval_hbm, vb0, sem0).wait()
            compute(0)

        for j, a in enumerate(accs):
            pltpu.async_copy(a, out_hbm.at[sp * n_bat + row0 + j], sem2)
        for j, a in enumerate(accs):
            pltpu.make_async_copy(a, out_hbm.at[sp * n_bat + row0 + j], sem2).wait()

    return spmm(xs, indices, values, tidx, tval, bias1d)


def _tc_reduce(partials, splits, n_bat, n_out):
    blk = 2048

    def body(p_ref, o_ref):
        o_ref[...] = jnp.sum(p_ref[...], axis=0)

    return pl.pallas_call(
        body,
        grid=(n_out // blk,),
        in_specs=[pl.BlockSpec((splits, n_bat, blk), lambda i: (0, 0, i))],
        out_specs=pl.BlockSpec((n_bat, blk), lambda i: (0, i)),
        out_shape=jax.ShapeDtypeStruct((n_bat, n_out), jnp.float32),
    )(partials)


def kernel(x, indices, values, bias):
    n_bat, n_in = x.shape[0], x.shape[1]
    n_out = bias.shape[0]
    nnz = values.shape[0]

    nbg = n_bat // R
    splits = NW // nbg

    xs = x.reshape(n_bat, n_in)
    bias1d = bias.reshape(n_out)

    n_full = nnz // CHUNK
    t = nnz - n_full * CHUNK
    tidx = jnp.pad(lax.slice(indices, (0, nnz - t), (2, nnz)),
                   ((0, 0), (0, CHUNK - t)))
    tval = jnp.pad(lax.slice(values, (nnz - t,), (nnz,)), (0, CHUNK - t))

    partials = _sc_partials(xs, indices, values, tidx, tval, bias1d,
                            n_bat, n_in, n_out)
    partials = partials.reshape(splits, n_bat, n_out)
    out = _tc_reduce(partials, splits, n_bat, n_out)
    return out[..., None]
